# baseline (speedup 1.0000x reference)
import functools
import numpy as np
import jax
import jax.numpy as jnp
from jax import lax
from jax.experimental import pallas as pl
from jax.experimental.pallas import tpu as pltpu

_EPS = float(np.finfo(np.float32).eps)


def _fused(x2_ref, x1t_ref, f1_ref, f2_ref, w0at_ref, w0bt_ref, w1t_ref,
           g0_ref, be0_ref, g1_ref, be1_ref, out_ref,
           y0_scr, y1_scr, s0_scr, ss0_scr, sc1_scr, sh1_scr,
           *, nb, count):
    p = pl.program_id(0)
    npts = x2_ref.shape[1]

    @pl.when(p < nb)
    def _phase_a():
        b = p
        x2 = x2_ref[0]
        x1t = x1t_ref[0]
        n = x1t.shape[1]

        acc = None
        for d in range(3):
            diff = x2[:, d:d + 1] - x1t[d:d + 1, :]
            sq = diff * diff
            acc = sq if acc is None else acc + sq
        r = 1.0 / (acc + _EPS)

        m1 = jnp.max(r, axis=1, keepdims=True)
        r2 = jnp.where(r == m1, 0.0, r)
        m2 = jnp.max(r2, axis=1, keepdims=True)
        r3 = jnp.where(r2 == m2, 0.0, r2)
        m3 = jnp.max(r3, axis=1, keepdims=True)
        wmat = jnp.where(r >= m3, r, 0.0)
        inv_tot = 1.0 / jnp.sum(wmat, axis=1, keepdims=True)

        interp = jnp.dot(wmat, f1_ref[0],
                         preferred_element_type=jnp.float32) * inv_tot
        y0 = jnp.dot(interp, w0at_ref[...], preferred_element_type=jnp.float32)
        y0 = y0 + jnp.dot(f2_ref[0], w0bt_ref[...],
                          preferred_element_type=jnp.float32)
        y0_scr[pl.ds(b * npts, npts)] = y0

        s = jnp.sum(y0, axis=0, keepdims=True)
        ss = jnp.sum(y0 * y0, axis=0, keepdims=True)

        @pl.when(b == 0)
        def _():
            s0_scr[...] = s
            ss0_scr[...] = ss

        @pl.when(b != 0)
        def _():
            s0_scr[...] += s
            ss0_scr[...] += ss

    @pl.when(p == nb)
    def _phase_b():
        mean = s0_scr[...] * (1.0 / count)
        var = ss0_scr[...] * (1.0 / count) - mean * mean
        inv = lax.rsqrt(var + 1e-5)
        scale = g0_ref[...] * inv
        shift = be0_ref[...] - mean * scale
        x = jnp.maximum(y0_scr[...] * scale + shift, 0.0)
        y1 = jnp.dot(x, w1t_ref[...], preferred_element_type=jnp.float32)
        y1_scr[...] = y1

        s1 = jnp.sum(y1, axis=0, keepdims=True)
        ss1 = jnp.sum(y1 * y1, axis=0, keepdims=True)
        mean1 = s1 * (1.0 / count)
        var1 = ss1 * (1.0 / count) - mean1 * mean1
        inv1 = lax.rsqrt(var1 + 1e-5)
        sc1_scr[...] = g1_ref[...] * inv1
        sh1_scr[...] = be1_ref[...] - mean1 * (g1_ref[...] * inv1)

    @pl.when(p > nb)
    def _phase_c():
        b = p - nb - 1
        out_ref[0] = jnp.maximum(
            y1_scr[pl.ds(b * npts, npts)] * sc1_scr[...] + sh1_scr[...], 0.0)


def kernel(xyz1, xyz2, features1, features2, W0, b0, g0, beta0,
           W1, b1, g1, beta1):
    B, N, _ = xyz1.shape
    Np = xyz2.shape[1]
    D1 = features1.shape[2]
    D2 = features2.shape[2]
    C0 = W0.shape[0]
    C1 = W1.shape[0]
    count = float(B * Np)

    x1t = xyz1.transpose(0, 2, 1)
    x2p = jnp.pad(xyz2, ((0, 0), (0, 0), (0, 13)))
    w0at = W0[:, :D1].T
    w0bt = W0[:, D1:].T
    w1t = W1.T
    g0r = g0.reshape(1, C0)
    beta0r = beta0.reshape(1, C0)
    g1r = g1.reshape(1, C1)
    beta1r = beta1.reshape(1, C1)

    def a_block(shape):
        return pl.BlockSpec(
            (1,) + shape,
            lambda p: (jnp.minimum(p, B - 1),) + (0,) * len(shape))

    def fixed_block(shape):
        return pl.BlockSpec(shape, lambda p: (0,) * len(shape))

    out = pl.pallas_call(
        functools.partial(_fused, nb=B, count=count),
        grid=(2 * B + 1,),
        in_specs=[
            a_block((Np, 16)),
            a_block((3, N)),
            a_block((N, D1)),
            a_block((Np, D2)),
            fixed_block((D1, C0)),
            fixed_block((D2, C0)),
            fixed_block((C0, C1)),
            fixed_block((1, C0)),
            fixed_block((1, C0)),
            fixed_block((1, C1)),
            fixed_block((1, C1)),
        ],
        out_specs=pl.BlockSpec(
            (1, Np, C1),
            lambda p: (jnp.maximum(p - B - 1, 0), 0, 0)),
        out_shape=jax.ShapeDtypeStruct((B, Np, C1), jnp.float32),
        scratch_shapes=[
            pltpu.VMEM((B * Np, C0), jnp.float32),
            pltpu.VMEM((B * Np, C1), jnp.float32),
            pltpu.VMEM((1, C0), jnp.float32),
            pltpu.VMEM((1, C0), jnp.float32),
            pltpu.VMEM((1, C1), jnp.float32),
            pltpu.VMEM((1, C1), jnp.float32),
        ],
        compiler_params=pltpu.CompilerParams(
            dimension_semantics=("arbitrary",)),
    )(x2p, x1t, features1, features2, w0at, w0bt, w1t,
      g0r, beta0r, g1r, beta1r)

    return out


# speedup vs baseline: 1.1305x; 1.1305x over previous
import functools
import numpy as np
import jax
import jax.numpy as jnp
from jax import lax
from jax.experimental import pallas as pl
from jax.experimental.pallas import tpu as pltpu

_EPS = float(np.finfo(np.float32).eps)


def _fused(x2_ref, x1t_ref, f1_ref, f2_ref, w0at_ref, w0bt_ref, w1t_ref,
           g0_ref, be0_ref, g1_ref, be1_ref, out_ref,
           y0_scr, y1_scr, s0_scr, ss0_scr, sc1_scr, sh1_scr,
           *, nb, count):
    p = pl.program_id(0)
    npts = x2_ref.shape[1]

    @pl.when(p < nb)
    def _phase_a():
        b = p
        x2 = x2_ref[0]
        x1t = x1t_ref[0]
        n = x1t.shape[1]

        acc = None
        for d in range(3):
            diff = x2[:, d:d + 1] - x1t[d:d + 1, :]
            sq = diff * diff
            acc = sq if acc is None else acc + sq

        inf = jnp.float32(np.inf)
        m1 = jnp.min(acc, axis=1, keepdims=True)
        d2 = jnp.where(acc == m1, inf, acc)
        m2 = jnp.min(d2, axis=1, keepdims=True)
        d3 = jnp.where(d2 == m2, inf, d2)
        m3 = jnp.min(d3, axis=1, keepdims=True)
        r1 = 1.0 / (m1 + _EPS)
        r2 = 1.0 / (m2 + _EPS)
        r3 = 1.0 / (m3 + _EPS)
        inv_tot = 1.0 / (r1 + r2 + r3)
        wmat = jnp.where(
            acc == m1, r1,
            jnp.where(acc == m2, r2, jnp.where(acc == m3, r3, 0.0)))

        interp = jnp.dot(wmat, f1_ref[0],
                         preferred_element_type=jnp.float32) * inv_tot
        y0 = jnp.dot(interp, w0at_ref[...], preferred_element_type=jnp.float32)
        y0 = y0 + jnp.dot(f2_ref[0], w0bt_ref[...],
                          preferred_element_type=jnp.float32)
        y0_scr[pl.ds(b * npts, npts)] = y0

        s = jnp.sum(y0, axis=0, keepdims=True)
        ss = jnp.sum(y0 * y0, axis=0, keepdims=True)

        @pl.when(b == 0)
        def _():
            s0_scr[...] = s
            ss0_scr[...] = ss

        @pl.when(b != 0)
        def _():
            s0_scr[...] += s
            ss0_scr[...] += ss

    @pl.when(p == nb)
    def _phase_b():
        mean = s0_scr[...] * (1.0 / count)
        var = ss0_scr[...] * (1.0 / count) - mean * mean
        inv = lax.rsqrt(var + 1e-5)
        scale = g0_ref[...] * inv
        shift = be0_ref[...] - mean * scale
        x = jnp.maximum(y0_scr[...] * scale + shift, 0.0)
        y1 = jnp.dot(x, w1t_ref[...], preferred_element_type=jnp.float32)
        y1_scr[...] = y1

        s1 = jnp.sum(y1, axis=0, keepdims=True)
        ss1 = jnp.sum(y1 * y1, axis=0, keepdims=True)
        mean1 = s1 * (1.0 / count)
        var1 = ss1 * (1.0 / count) - mean1 * mean1
        inv1 = lax.rsqrt(var1 + 1e-5)
        sc1_scr[...] = g1_ref[...] * inv1
        sh1_scr[...] = be1_ref[...] - mean1 * (g1_ref[...] * inv1)

    @pl.when(p > nb)
    def _phase_c():
        b = p - nb - 1
        out_ref[0] = jnp.maximum(
            y1_scr[pl.ds(b * npts, npts)] * sc1_scr[...] + sh1_scr[...], 0.0)


def kernel(xyz1, xyz2, features1, features2, W0, b0, g0, beta0,
           W1, b1, g1, beta1):
    B, N, _ = xyz1.shape
    Np = xyz2.shape[1]
    D1 = features1.shape[2]
    D2 = features2.shape[2]
    C0 = W0.shape[0]
    C1 = W1.shape[0]
    count = float(B * Np)

    x1t = xyz1.transpose(0, 2, 1)
    w0at = W0[:, :D1].T
    w0bt = W0[:, D1:].T
    w1t = W1.T
    g0r = g0.reshape(1, C0)
    beta0r = beta0.reshape(1, C0)
    g1r = g1.reshape(1, C1)
    beta1r = beta1.reshape(1, C1)

    def a_block(shape):
        return pl.BlockSpec(
            (1,) + shape,
            lambda p: (jnp.minimum(p, B - 1),) + (0,) * len(shape))

    def fixed_block(shape):
        return pl.BlockSpec(shape, lambda p: (0,) * len(shape))

    out = pl.pallas_call(
        functools.partial(_fused, nb=B, count=count),
        grid=(2 * B + 1,),
        in_specs=[
            a_block((Np, 3)),
            a_block((3, N)),
            a_block((N, D1)),
            a_block((Np, D2)),
            fixed_block((D1, C0)),
            fixed_block((D2, C0)),
            fixed_block((C0, C1)),
            fixed_block((1, C0)),
            fixed_block((1, C0)),
            fixed_block((1, C1)),
            fixed_block((1, C1)),
        ],
        out_specs=pl.BlockSpec(
            (1, Np, C1),
            lambda p: (jnp.maximum(p - B - 1, 0), 0, 0)),
        out_shape=jax.ShapeDtypeStruct((B, Np, C1), jnp.float32),
        scratch_shapes=[
            pltpu.VMEM((B * Np, C0), jnp.float32),
            pltpu.VMEM((B * Np, C1), jnp.float32),
            pltpu.VMEM((1, C0), jnp.float32),
            pltpu.VMEM((1, C0), jnp.float32),
            pltpu.VMEM((1, C1), jnp.float32),
            pltpu.VMEM((1, C1), jnp.float32),
        ],
        compiler_params=pltpu.CompilerParams(
            dimension_semantics=("arbitrary",)),
    )(xyz2, x1t, features1, features2, w0at, w0bt, w1t,
      g0r, beta0r, g1r, beta1r)

    return out
